# baseline (speedup 1.0000x reference)
import functools

import jax
import jax.numpy as jnp
from jax import lax
from jax.experimental import pallas as pl
from jax.experimental.pallas import tpu as pltpu
from jax.experimental.pallas import tpu_sc as plsc

_NC = 2
_NS = 16
_NW = _NC * _NS
_BLK = 128
_PAIRS = 2
_CHUNK = _PAIRS * _BLK
_NBUF = 2


@functools.cache
def _build(batch, seq, vocab, dim):
    n_bblk = batch // _BLK
    n_pairs = seq * n_bblk
    pairs_per_worker = n_pairs // _NW
    chunks_per_worker = pairs_per_worker // _PAIRS
    assert chunks_per_worker % _NBUF == 0
    n_dblk = dim // 8
    s_bytes = n_dblk * n_bblk * 8 * _BLK
    tile_words = _PAIRS * 8 * _BLK

    mesh = plsc.VectorSubcoreMesh(core_axis_name="c", subcore_axis_name="s")

    @functools.partial(
        pl.kernel,
        out_type=jax.ShapeDtypeStruct((seq, s_bytes), jnp.float32),
        mesh=mesh,
        compiler_params=pltpu.CompilerParams(
            use_tc_tiling_on_sc=False, needs_layout_passes=False),
        scratch_types=[
            pltpu.VMEM((_NBUF, _CHUNK), jnp.int32),
            pltpu.VMEM((_NBUF, _CHUNK, dim), jnp.float32),
            pltpu.VMEM((_NBUF, n_dblk * tile_words), jnp.float32),
            pltpu.SemaphoreType.DMA,
            pltpu.SemaphoreType.DMA,
            pltpu.SemaphoreType.DMA,
            pltpu.SemaphoreType.DMA,
        ],
    )
    def gather_kernel(text_hbm, table_hbm, out_hbm, idx_v, rows_v, tile_v,
                      gsem0, gsem1, osem0, osem1):
        c = lax.axis_index("c")
        s_ax = lax.axis_index("s")
        wid = s_ax * _NC + c
        pair0 = wid * pairs_per_worker
        gsems = (gsem0, gsem1)
        osems = (osem0, osem1)
        iota16 = jax.lax.iota(jnp.int32, 16)

        def chunk_pos(g):
            p = pair0 + g * _PAIRS
            return p // n_bblk, p % n_bblk

        def start_chunk(g, b):
            s, bt0 = chunk_pos(g)
            pltpu.sync_copy(text_hbm.at[s, pl.ds(bt0 * _BLK, _CHUNK)],
                            idx_v.at[b])
            for j in range(_PAIRS):
                pltpu.async_copy(
                    table_hbm.at[idx_v.at[b, pl.ds(j * _BLK, _BLK)]],
                    rows_v.at[b, pl.ds(j * _BLK, _BLK)],
                    gsems[b])

        def wait_gather(b):
            for j in range(_PAIRS):
                pltpu.make_async_copy(
                    table_hbm.at[idx_v.at[b, pl.ds(j * _BLK, _BLK)]],
                    rows_v.at[b, pl.ds(j * _BLK, _BLK)],
                    gsems[b]).wait()

        def out_copies(g, b, fire):
            s, bt0 = chunk_pos(g)
            for dt in range(n_dblk):
                desc = pltpu.make_async_copy(
                    tile_v.at[b, pl.ds(dt * tile_words, tile_words)],
                    out_hbm.at[s, pl.ds((dt * n_bblk + bt0) * 8 * _BLK,
                                        tile_words)],
                    osems[b])
                desc.start() if fire else desc.wait()

        def transpose_chunk(b):
            def t_body(k, carry):
                for p in range(_PAIRS):
                    for u in range(8):
                        d = k * 8 + u
                        for b16 in range(8):
                            row_idx = iota16 + (p * _BLK + b16 * 16)
                            col_idx = jnp.full((16,), u, jnp.int32) + k * 8
                            vals = plsc.load_gather(
                                rows_v.at[b], [row_idx, col_idx])
                            tile_v[b, pl.ds(
                                k * 2048 + (p * 1024 + u * 128 + b16 * 16),
                                16)] = vals
                return carry

            lax.fori_loop(0, n_dblk, t_body, 0)

        for b in range(_NBUF):
            start_chunk(b, b)

        def loop_body(t, carry):
            for b in range(_NBUF):
                g = t * _NBUF + b
                wait_gather(b)

                @pl.when(g >= _NBUF)
                def _():
                    out_copies(g - _NBUF, b, fire=False)

                transpose_chunk(b)
                out_copies(g, b, fire=True)

                @pl.when(g + _NBUF < chunks_per_worker)
                def _():
                    start_chunk(g + _NBUF, b)
            return carry

        lax.fori_loop(0, chunks_per_worker // _NBUF, loop_body, 0)

        for b in range(_NBUF):
            out_copies(chunks_per_worker - _NBUF + b, b, fire=False)

    return gather_kernel


def kernel(text, table):
    batch, seq = text.shape
    vocab, dim = table.shape
    text_t = jnp.transpose(text).astype(jnp.int32)
    out2 = _build(batch, seq, vocab, dim)(text_t, table)
    n_bblk = batch // _BLK
    n_dblk = dim // 8
    out6 = out2.reshape(seq, n_dblk, n_bblk, 8, _BLK)
    return jnp.transpose(out6, (2, 4, 0, 1, 3)).reshape(batch, seq, dim)


# speedup vs baseline: 1.9667x; 1.9667x over previous
import functools

import jax
import jax.numpy as jnp
from jax import lax
from jax.experimental import pallas as pl
from jax.experimental.pallas import tpu as pltpu
from jax.experimental.pallas import tpu_sc as plsc

_NC = 2
_NS = 16
_NW = _NC * _NS
_BLK = 128
_PAIRS = 2
_CHUNK = _PAIRS * _BLK
_NBUF = 2
_LPAD = _BLK + 1


@functools.cache
def _build(batch, seq, vocab, dim):
    n_bblk = batch // _BLK
    n_pairs = seq * n_bblk
    pairs_per_worker = n_pairs // _NW
    chunks_per_worker = pairs_per_worker // _PAIRS
    assert chunks_per_worker % _NBUF == 0
    idx_per_worker = pairs_per_worker * _BLK
    n_dblk = dim // 8
    tile_rows = _PAIRS * 8

    mesh = plsc.VectorSubcoreMesh(core_axis_name="c", subcore_axis_name="s")

    @functools.partial(
        pl.kernel,
        out_type=jax.ShapeDtypeStruct((seq, n_dblk * n_bblk * 8, _BLK),
                                      jnp.float32),
        mesh=mesh,
        compiler_params=pltpu.CompilerParams(
            use_tc_tiling_on_sc=False, needs_layout_passes=False),
        scratch_types=[
            pltpu.VMEM((idx_per_worker,), jnp.int32),
            pltpu.VMEM((_NBUF, _CHUNK, dim), jnp.float32),
            pltpu.VMEM((_NBUF, n_dblk, tile_rows, _LPAD), jnp.float32),
            pltpu.SemaphoreType.DMA,
            pltpu.SemaphoreType.DMA,
            pltpu.SemaphoreType.DMA,
            pltpu.SemaphoreType.DMA,
        ],
    )
    def gather_kernel(text_hbm, table_hbm, out_hbm, idx_v, rows_v, tile_v,
                      gsem0, gsem1, osem0, osem1):
        c = lax.axis_index("c")
        s_ax = lax.axis_index("s")
        wid = s_ax * _NC + c
        pair0 = wid * pairs_per_worker
        gsems = (gsem0, gsem1)
        osems = (osem0, osem1)
        iota16 = jax.lax.iota(jnp.int32, 16)

        pltpu.sync_copy(text_hbm.at[pl.ds(pair0 * _BLK, idx_per_worker)],
                        idx_v)

        def chunk_pos(g):
            p = pair0 + g * _PAIRS
            return p // n_bblk, p % n_bblk

        def gather_copies(g, b, fire):
            for j in range(_PAIRS):
                desc = pltpu.make_async_copy(
                    table_hbm.at[idx_v.at[pl.ds((g * _PAIRS + j) * _BLK,
                                                _BLK)]],
                    rows_v.at[b, pl.ds(j * _BLK, _BLK)],
                    gsems[b])
                desc.start() if fire else desc.wait()

        def out_copies(g, b, fire):
            s, bt0 = chunk_pos(g)
            for dt in range(n_dblk):
                desc = pltpu.make_async_copy(
                    tile_v.at[b, dt, pl.ds(0, tile_rows), pl.ds(0, _BLK)],
                    out_hbm.at[s, pl.ds(dt * n_bblk * 8 + bt0 * 8,
                                        tile_rows)],
                    osems[b])
                desc.start() if fire else desc.wait()

        dt_vecs = [(iota16 // 8) + 2 * cc for cc in range(dim // 16)]
        sub_vec = iota16 % 8

        def transpose_chunk(b):
            def t_body(t, carry):
                p = t // (_BLK // 8)
                lbase = (t % (_BLK // 8)) * 8
                row_vec = sub_vec + p * 8
                for rr in range(8):
                    lane = lbase + rr
                    r = p * _BLK + lane
                    lane_vec = jnp.full((16,), 0, jnp.int32) + lane
                    for cc in range(dim // 16):
                        vals = rows_v[b, r, pl.ds(cc * 16, 16)]
                        plsc.store_scatter(
                            tile_v.at[b],
                            [dt_vecs[cc], row_vec, lane_vec], vals)
                return carry

            lax.fori_loop(0, _CHUNK // 8, t_body, 0)

        for b in range(_NBUF):
            gather_copies(b, b, fire=True)

        def loop_body(t, carry):
            for b in range(_NBUF):
                g = t * _NBUF + b
                gather_copies(g, b, fire=False)

                @pl.when(g >= _NBUF)
                def _():
                    out_copies(g - _NBUF, b, fire=False)

                transpose_chunk(b)
                out_copies(g, b, fire=True)

                @pl.when(g + _NBUF < chunks_per_worker)
                def _():
                    gather_copies(g + _NBUF, b, fire=True)
            return carry

        lax.fori_loop(0, chunks_per_worker // _NBUF, loop_body, 0)

        for b in range(_NBUF):
            out_copies(chunks_per_worker - _NBUF + b, b, fire=False)

    return gather_kernel


def kernel(text, table):
    batch, seq = text.shape
    vocab, dim = table.shape
    text_flat = jnp.transpose(text).astype(jnp.int32).reshape(batch * seq)
    out3 = _build(batch, seq, vocab, dim)(text_flat, table)
    n_bblk = batch // _BLK
    n_dblk = dim // 8
    out6 = out3.reshape(seq, n_dblk, n_bblk, 8, _BLK)
    return jnp.transpose(out6, (2, 4, 0, 1, 3)).reshape(batch, seq, dim)
